# Initial kernel scaffold; baseline (speedup 1.0000x reference)
import jax
import jax.numpy as jnp
from jax.experimental import pallas as pl


def kernel(x, emb_table, lin_weight, out_bias):
    raise NotImplementedError("write your pallas kernel here")


# speedup vs baseline: 102.6592x; 102.6592x over previous
import functools

import jax
import jax.numpy as jnp
import numpy as np
from jax import lax
from jax.experimental import pallas as pl
from jax.experimental.pallas import tpu as pltpu
from jax.experimental.pallas import tpu_sc as plsc

_NUM_TREES = 600
_EMB_DIM = 3
_VOCAB = _NUM_TREES * 1024
_BATCH = 16384
_LANES = 16

_NW = 32
_ROWS_PER_W = _BATCH // _NW
_CB = 8
_NCHUNK = _ROWS_PER_W // _CB
_ROW_ELEMS = _NUM_TREES * _EMB_DIM
_NVEC = (_ROW_ELEMS + _LANES - 1) // _LANES
_PAT = _NVEC * _LANES

_f = np.minimum(np.arange(_PAT, dtype=np.int64), _ROW_ELEMS - 1)
_ROWPAT = np.asarray(_f // _EMB_DIM, dtype=np.int32)
_COLPAT = np.asarray(_f % _EMB_DIM, dtype=np.int32)


def _body(xf_hbm, table_hbm, rowpat_hbm, colpat_hbm, wpad_hbm, bias_hbm,
          out_hbm,
          idx_v, rows_v, rowpat_v, colpat_v, wpad_v, bias_v, outbuf_v, sem):
    wid = lax.axis_index("s") * 2 + lax.axis_index("c")
    pltpu.sync_copy(rowpat_hbm, rowpat_v)
    pltpu.sync_copy(colpat_hbm, colpat_v)
    pltpu.sync_copy(wpad_hbm, wpad_v)
    pltpu.sync_copy(bias_hbm, bias_v)
    base_row = wid * _ROWS_PER_W
    lane0 = lax.iota(jnp.int32, _LANES) == 0

    def chunk_body(c, carry):
        src = (base_row + c * _CB) * _NUM_TREES
        pltpu.sync_copy(xf_hbm.at[pl.ds(src, _CB * _NUM_TREES)], idx_v)
        pltpu.async_copy(table_hbm.at[idx_v], rows_v, sem).wait()
        bias = bias_v[...]

        def j_body(j, accs):
            jo = j * _LANES
            rp = rowpat_v[pl.ds(jo, _LANES)]
            cv = colpat_v[pl.ds(jo, _LANES)]
            w = wpad_v[pl.ds(jo, _LANES)]
            return tuple(
                accs[r] + plsc.load_gather(rows_v, [rp + (r * _NUM_TREES), cv]) * w
                for r in range(_CB))

        accs = lax.fori_loop(0, _NVEC, j_body,
                             tuple(bias for _ in range(_CB)))
        for r in range(_CB):
            s = jnp.sum(accs[r])
            pos = jnp.full((_LANES,), c * _CB + r, dtype=jnp.int32)
            val = jnp.full((_LANES,), s, dtype=jnp.float32)
            plsc.store_scatter(outbuf_v, [pos], val, mask=lane0)
        return carry

    lax.fori_loop(0, _NCHUNK, chunk_body, 0)
    pltpu.sync_copy(outbuf_v, out_hbm.at[pl.ds(base_row, _ROWS_PER_W)])


def kernel(x, emb_table, lin_weight, out_bias):
    xf = x.reshape(-1)
    wflat = lin_weight.reshape(-1)
    wpad = jnp.concatenate(
        [wflat, jnp.zeros((_PAT - _ROW_ELEMS,), jnp.float32)])
    bias_v = jnp.zeros((_LANES,), jnp.float32).at[0].set(out_bias)
    rowpat = jnp.asarray(_ROWPAT)
    colpat = jnp.asarray(_COLPAT)

    mesh = plsc.VectorSubcoreMesh(core_axis_name="c", subcore_axis_name="s")
    run = pl.kernel(
        _body,
        mesh=mesh,
        compiler_params=pltpu.CompilerParams(needs_layout_passes=False,
                                             use_tc_tiling_on_sc=False),
        out_type=jax.ShapeDtypeStruct((_BATCH,), jnp.float32),
        scratch_types=[
            pltpu.VMEM((_CB * _NUM_TREES,), jnp.int32),
            pltpu.VMEM((_CB * _NUM_TREES, _EMB_DIM), jnp.float32),
            pltpu.VMEM((_PAT,), jnp.int32),
            pltpu.VMEM((_PAT,), jnp.int32),
            pltpu.VMEM((_PAT,), jnp.float32),
            pltpu.VMEM((_LANES,), jnp.float32),
            pltpu.VMEM((_ROWS_PER_W,), jnp.float32),
            pltpu.SemaphoreType.DMA,
        ],
    )
    return run(xf, emb_table, rowpat, colpat, wpad, bias_v)


# speedup vs baseline: 103.1189x; 1.0045x over previous
import jax
import jax.numpy as jnp
from jax import lax
from jax.experimental import pallas as pl
from jax.experimental.pallas import tpu as pltpu
from jax.experimental.pallas import tpu_sc as plsc

_NUM_TREES = 600
_EMB_DIM = 3
_PAD_DIM = 4
_BATCH = 16384
_LANES = 16

_NW = 32
_ROWS_PER_W = _BATCH // _NW
_CB = 8
_NCHUNK = _ROWS_PER_W // _CB
_ROW_ELEMS = _NUM_TREES * _PAD_DIM
_NVEC = _ROW_ELEMS // _LANES
_NBUF = 1


def _body(x_hbm, table_hbm, wpad_hbm, bias_hbm, out_hbm,
          idx_v, rows_v, wpad_v, bias_v, outbuf_v, sem0, sem1):
    sems = (sem0, sem1)
    wid = lax.axis_index("s") * 2 + lax.axis_index("c")
    pltpu.sync_copy(wpad_hbm, wpad_v)
    pltpu.sync_copy(bias_hbm, bias_v)
    base_row = wid * _ROWS_PER_W
    lane = lax.iota(jnp.int32, _LANES)
    lane0 = lane == 0
    lane_q = lax.shift_right_logical(lane, 2)

    def fire(c, p):
        pltpu.sync_copy(x_hbm.at[pl.ds(base_row + c * _CB, _CB), :],
                        idx_v.at[p])
        for r in range(_CB):
            pltpu.make_async_copy(
                table_hbm.at[idx_v.at[p].at[r]],
                rows_v.at[p].at[pl.ds(r * _NUM_TREES, _NUM_TREES), :],
                sems[p],
            ).start()

    def drain(p):
        for r in range(_CB):
            pltpu.make_async_copy(
                table_hbm.at[idx_v.at[p].at[r]],
                rows_v.at[p].at[pl.ds(r * _NUM_TREES, _NUM_TREES), :],
                sems[p],
            ).wait()

    def compute(c, p):
        bias = bias_v[...]
        rows_p = rows_v.at[p]

        def j_body(j, accs):
            w = wpad_v[pl.ds(j * _LANES, _LANES)]
            rbase = lane_q + j * _PAD_DIM
            new = []
            for r in range(_CB):
                g = plsc.load_gather(
                    rows_p, [rbase + (r * _NUM_TREES), lane & 3])
                new.append(accs[r] + g * w)
            return tuple(new)

        accs = lax.fori_loop(0, _NVEC, j_body,
                             tuple(bias for _ in range(_CB)))
        for r in range(_CB):
            s = jnp.sum(accs[r])
            pos = jnp.full((_LANES,), c * _CB + r, dtype=jnp.int32)
            val = jnp.full((_LANES,), s, dtype=jnp.float32)
            plsc.store_scatter(outbuf_v, [pos], val, mask=lane0)

    for p in range(_NBUF):
        fire(p, p)

    def outer(c0, carry):
        for p in range(_NBUF):
            c = c0 + p
            drain(p)
            compute(c, p)

            @pl.when(c + _NBUF < _NCHUNK)
            def _():
                fire(c + _NBUF, p)
        return carry

    lax.fori_loop(0, _NCHUNK // _NBUF, lambda i, cr: outer(i * _NBUF, cr), 0)
    pltpu.sync_copy(outbuf_v, out_hbm.at[pl.ds(base_row, _ROWS_PER_W)])


def kernel(x, emb_table, lin_weight, out_bias):
    table4 = jnp.pad(emb_table, ((0, 0), (0, _PAD_DIM - _EMB_DIM)))
    w4 = jnp.pad(lin_weight.reshape(_NUM_TREES, _EMB_DIM),
                 ((0, 0), (0, _PAD_DIM - _EMB_DIM))).reshape(-1)
    bias_v = jnp.zeros((_LANES,), jnp.float32).at[0].set(out_bias)

    mesh = plsc.VectorSubcoreMesh(core_axis_name="c", subcore_axis_name="s")
    run = pl.kernel(
        _body,
        mesh=mesh,
        compiler_params=pltpu.CompilerParams(needs_layout_passes=False,
                                             use_tc_tiling_on_sc=False),
        out_type=jax.ShapeDtypeStruct((_BATCH,), jnp.float32),
        scratch_types=[
            pltpu.VMEM((_NBUF, _CB, _NUM_TREES), jnp.int32),
            pltpu.VMEM((_NBUF, _CB * _NUM_TREES, _PAD_DIM), jnp.float32),
            pltpu.VMEM((_ROW_ELEMS,), jnp.float32),
            pltpu.VMEM((_LANES,), jnp.float32),
            pltpu.VMEM((_ROWS_PER_W,), jnp.float32),
            pltpu.SemaphoreType.DMA,
            pltpu.SemaphoreType.DMA,
        ],
    )
    return run(x, table4, w4, bias_v)


# speedup vs baseline: 115.8296x; 1.1233x over previous
import jax
import jax.numpy as jnp
import numpy as np
from jax import lax
from jax.experimental import pallas as pl
from jax.experimental.pallas import tpu as pltpu
from jax.experimental.pallas import tpu_sc as plsc

_NUM_TREES = 600
_EMB_DIM = 3
_VOCAB = _NUM_TREES * 1024
_BATCH = 16384
_LANES = 16

_NW = 32
_NSUB = 16
_ROWS_PER_W = _BATCH // _NW
_CB = 8
_NCHUNK = _ROWS_PER_W // _CB
_ROW_ELEMS = _NUM_TREES * _EMB_DIM
_NVEC = (_ROW_ELEMS + _LANES - 1) // _LANES
_PAT = _NVEC * _LANES
_NBUF = 1

_FCH = 4
_FROW = _VOCAB // (_NW * _FCH)
_FEL = _FROW * _EMB_DIM
_FQ = _FEL // (3 * _LANES)

_f = np.minimum(np.arange(_PAT, dtype=np.int64), _ROW_ELEMS - 1)
_ROWPAT = np.asarray(_f // _EMB_DIM, dtype=np.int32)
_COLPAT = np.asarray(_f % _EMB_DIM, dtype=np.int32)


def _fmt_body(t1d_hbm, tout_hbm, buf1_v, rows3_v):
    wid = lax.axis_index("s") * 2 + lax.axis_index("c")
    lane = lax.iota(jnp.int32, _LANES)
    rpat = [(m * _LANES + lane) // 3 for m in range(3)]
    cpat = [(m * _LANES + lane) % 3 for m in range(3)]
    for h in range(_FCH):
        g = wid * _FCH + h
        pltpu.sync_copy(t1d_hbm.at[pl.ds(g * _FEL, _FEL)], buf1_v)

        def fill_q(q, carry):
            for m in range(3):
                v = buf1_v[pl.ds((q * 3 + m) * _LANES, _LANES)]
                plsc.store_scatter(rows3_v, [q * _LANES + rpat[m], cpat[m]], v)
            return carry

        lax.fori_loop(0, _FQ, fill_q, 0)
        pltpu.sync_copy(rows3_v, tout_hbm.at[pl.ds(g * _FROW, _FROW), :])


def _body(x_hbm, spt, rowpat_hbm, colpat_hbm, wpad_hbm, bias_hbm,
          out_hbm,
          idx_v, rows_v, rowpat_v, colpat_v, wpad_v,
          bias_v, outbuf_v, sem0, sem1):
    sems = (sem0, sem1)
    wid = lax.axis_index("s") * 2 + lax.axis_index("c")
    lane = lax.iota(jnp.int32, _LANES)
    lane0 = lane == 0

    pltpu.sync_copy(rowpat_hbm, rowpat_v)
    pltpu.sync_copy(colpat_hbm, colpat_v)
    pltpu.sync_copy(wpad_hbm, wpad_v)
    pltpu.sync_copy(bias_hbm, bias_v)
    base_row = wid * _ROWS_PER_W

    def fire(c, p):
        pltpu.sync_copy(x_hbm.at[pl.ds(base_row + c * _CB, _CB), :],
                        idx_v.at[p])
        for r in range(_CB):
            pltpu.make_async_copy(
                spt.at[idx_v.at[p].at[r]],
                rows_v.at[p].at[pl.ds(r * _NUM_TREES, _NUM_TREES), :],
                sems[p],
            ).start()

    def drain(p):
        for r in range(_CB):
            pltpu.make_async_copy(
                spt.at[idx_v.at[p].at[r]],
                rows_v.at[p].at[pl.ds(r * _NUM_TREES, _NUM_TREES), :],
                sems[p],
            ).wait()

    def compute(c, p):
        bias = bias_v[...]
        rows_p = rows_v.at[p]

        def j_body(j, accs):
            jo = j * _LANES
            rp = rowpat_v[pl.ds(jo, _LANES)]
            cv = colpat_v[pl.ds(jo, _LANES)]
            w = wpad_v[pl.ds(jo, _LANES)]
            new = []
            for r in range(_CB):
                g = plsc.load_gather(rows_p, [rp + (r * _NUM_TREES), cv])
                new.append(accs[r] + g * w)
            return tuple(new)

        accs = lax.fori_loop(0, _NVEC, j_body,
                             tuple(bias for _ in range(_CB)))
        for r in range(_CB):
            s = jnp.sum(accs[r])
            pos = jnp.full((_LANES,), c * _CB + r, dtype=jnp.int32)
            val = jnp.full((_LANES,), s, dtype=jnp.float32)
            plsc.store_scatter(outbuf_v, [pos], val, mask=lane0)

    for p in range(_NBUF):
        fire(p, p)

    def outer(c0, carry):
        for p in range(_NBUF):
            c = c0 + p
            drain(p)
            compute(c, p)

            @pl.when(c + _NBUF < _NCHUNK)
            def _():
                fire(c + _NBUF, p)
        return carry

    lax.fori_loop(0, _NCHUNK // _NBUF, lambda i, cr: outer(i * _NBUF, cr), 0)
    pltpu.sync_copy(outbuf_v, out_hbm.at[pl.ds(base_row, _ROWS_PER_W)])


def kernel(x, emb_table, lin_weight, out_bias):
    tflat = emb_table.reshape(-1)
    wpad = jnp.concatenate(
        [lin_weight.reshape(-1), jnp.zeros((_PAT - _ROW_ELEMS,), jnp.float32)])
    bias_v = jnp.zeros((_LANES,), jnp.float32).at[0].set(out_bias)
    rowpat = jnp.asarray(_ROWPAT)
    colpat = jnp.asarray(_COLPAT)

    mesh = plsc.VectorSubcoreMesh(core_axis_name="c", subcore_axis_name="s")
    fmt = pl.kernel(
        _fmt_body,
        mesh=mesh,
        compiler_params=pltpu.CompilerParams(needs_layout_passes=False,
                                             use_tc_tiling_on_sc=False),
        out_type=pltpu.HBM((_VOCAB, _EMB_DIM), jnp.float32),
        scratch_types=[
            pltpu.VMEM((_FEL,), jnp.float32),
            pltpu.VMEM((_FROW, _EMB_DIM), jnp.float32),
        ],
    )
    table_lin = fmt(tflat)
    run = pl.kernel(
        _body,
        mesh=mesh,
        compiler_params=pltpu.CompilerParams(needs_layout_passes=False,
                                             use_tc_tiling_on_sc=False),
        out_type=jax.ShapeDtypeStruct((_BATCH,), jnp.float32),
        scratch_types=[
            pltpu.VMEM((_NBUF, _CB, _NUM_TREES), jnp.int32),
            pltpu.VMEM((_NBUF, _CB * _NUM_TREES, _EMB_DIM), jnp.float32),
            pltpu.VMEM((_PAT,), jnp.int32),
            pltpu.VMEM((_PAT,), jnp.int32),
            pltpu.VMEM((_PAT,), jnp.float32),
            pltpu.VMEM((_LANES,), jnp.float32),
            pltpu.VMEM((_ROWS_PER_W,), jnp.float32),
            pltpu.SemaphoreType.DMA,
            pltpu.SemaphoreType.DMA,
        ],
    )
    return run(x, table_lin, rowpat, colpat, wpad, bias_v)


# speedup vs baseline: 115.9003x; 1.0006x over previous
import jax
import jax.numpy as jnp
import numpy as np
from jax import lax
from jax.experimental import pallas as pl
from jax.experimental.pallas import tpu as pltpu
from jax.experimental.pallas import tpu_sc as plsc

_NUM_TREES = 600
_EMB_DIM = 3
_VOCAB = _NUM_TREES * 1024
_BATCH = 16384
_LANES = 16

_NW = 32
_NSUB = 16
_ROWS_PER_W = _BATCH // _NW
_CB = 8
_NCHUNK = _ROWS_PER_W // _CB
_ROW_ELEMS = _NUM_TREES * _EMB_DIM
_NVEC = (_ROW_ELEMS + _LANES - 1) // _LANES
_PAT = _NVEC * _LANES
_NBUF = 1

_FCH = 4
_FROW = _VOCAB // (_NW * _FCH)
_FEL = _FROW * _EMB_DIM
_FQ = _FEL // (3 * _LANES)

_f = np.minimum(np.arange(_PAT, dtype=np.int64), _ROW_ELEMS - 1)
_ROWPAT = np.asarray(_f // _EMB_DIM, dtype=np.int32)
_COLPAT = np.asarray(_f % _EMB_DIM, dtype=np.int32)


def _fmt_body(t1d_hbm, tout_hbm, buf1_v, rows3_v):
    wid = lax.axis_index("s") * 2 + lax.axis_index("c")
    lane = lax.iota(jnp.int32, _LANES)
    rpat = [(m * _LANES + lane) // 3 for m in range(3)]
    cpat = [(m * _LANES + lane) % 3 for m in range(3)]
    for h in range(_FCH):
        g = wid * _FCH + h
        pltpu.sync_copy(t1d_hbm.at[pl.ds(g * _FEL, _FEL)], buf1_v)

        def fill_q(q, carry):
            for m in range(3):
                v = buf1_v[pl.ds((q * 3 + m) * _LANES, _LANES)]
                plsc.store_scatter(rows3_v, [q * _LANES + rpat[m], cpat[m]], v)
            return carry

        lax.fori_loop(0, _FQ, fill_q, 0)
        pltpu.sync_copy(rows3_v, tout_hbm.at[pl.ds(g * _FROW, _FROW), :])


def _body(xf_hbm, spt, rowpat_hbm, colpat_hbm, wpad_hbm, bias_hbm,
          out_hbm,
          idx_v, rows_v, rowpat_v, colpat_v, wpad_v,
          bias_v, outbuf_v, sem0, sem1):
    sems = (sem0, sem1)
    wid = lax.axis_index("s") * 2 + lax.axis_index("c")
    lane = lax.iota(jnp.int32, _LANES)
    lane0 = lane == 0

    pltpu.sync_copy(rowpat_hbm, rowpat_v)
    pltpu.sync_copy(colpat_hbm, colpat_v)
    pltpu.sync_copy(wpad_hbm, wpad_v)
    pltpu.sync_copy(bias_hbm, bias_v)
    base_row = wid * _ROWS_PER_W

    def fire(c, p):
        pltpu.sync_copy(
            xf_hbm.at[pl.ds((base_row + c * _CB) * _NUM_TREES,
                            _CB * _NUM_TREES)],
            idx_v.at[p])
        pltpu.make_async_copy(
            spt.at[idx_v.at[p]], rows_v.at[p], sems[p]).start()

    def drain(p):
        pltpu.make_async_copy(
            spt.at[idx_v.at[p]], rows_v.at[p], sems[p]).wait()

    def compute(c, p):
        bias = bias_v[...]
        rows_p = rows_v.at[p]

        def j_body(j, accs):
            jo = j * _LANES
            rp = rowpat_v[pl.ds(jo, _LANES)]
            cv = colpat_v[pl.ds(jo, _LANES)]
            w = wpad_v[pl.ds(jo, _LANES)]
            new = []
            for r in range(_CB):
                g = plsc.load_gather(rows_p, [rp + (r * _NUM_TREES), cv])
                new.append(accs[r] + g * w)
            return tuple(new)

        accs = lax.fori_loop(0, _NVEC, j_body,
                             tuple(bias for _ in range(_CB)))
        for r in range(_CB):
            s = jnp.sum(accs[r])
            pos = jnp.full((_LANES,), c * _CB + r, dtype=jnp.int32)
            val = jnp.full((_LANES,), s, dtype=jnp.float32)
            plsc.store_scatter(outbuf_v, [pos], val, mask=lane0)

    for p in range(_NBUF):
        fire(p, p)

    def outer(c0, carry):
        for p in range(_NBUF):
            c = c0 + p
            drain(p)
            compute(c, p)

            @pl.when(c + _NBUF < _NCHUNK)
            def _():
                fire(c + _NBUF, p)
        return carry

    lax.fori_loop(0, _NCHUNK // _NBUF, lambda i, cr: outer(i * _NBUF, cr), 0)
    pltpu.sync_copy(outbuf_v, out_hbm.at[pl.ds(base_row, _ROWS_PER_W)])


def kernel(x, emb_table, lin_weight, out_bias):
    tflat = emb_table.reshape(-1)
    wpad = jnp.concatenate(
        [lin_weight.reshape(-1), jnp.zeros((_PAT - _ROW_ELEMS,), jnp.float32)])
    bias_v = jnp.zeros((_LANES,), jnp.float32).at[0].set(out_bias)
    rowpat = jnp.asarray(_ROWPAT)
    colpat = jnp.asarray(_COLPAT)

    mesh = plsc.VectorSubcoreMesh(core_axis_name="c", subcore_axis_name="s")
    fmt = pl.kernel(
        _fmt_body,
        mesh=mesh,
        compiler_params=pltpu.CompilerParams(needs_layout_passes=False,
                                             use_tc_tiling_on_sc=False),
        out_type=pltpu.HBM((_VOCAB, _EMB_DIM), jnp.float32),
        scratch_types=[
            pltpu.VMEM((_FEL,), jnp.float32),
            pltpu.VMEM((_FROW, _EMB_DIM), jnp.float32),
        ],
    )
    table_lin = fmt(tflat)
    run = pl.kernel(
        _body,
        mesh=mesh,
        compiler_params=pltpu.CompilerParams(needs_layout_passes=False,
                                             use_tc_tiling_on_sc=False),
        out_type=jax.ShapeDtypeStruct((_BATCH,), jnp.float32),
        scratch_types=[
            pltpu.VMEM((_NBUF, _CB * _NUM_TREES), jnp.int32),
            pltpu.VMEM((_NBUF, _CB * _NUM_TREES, _EMB_DIM), jnp.float32),
            pltpu.VMEM((_PAT,), jnp.int32),
            pltpu.VMEM((_PAT,), jnp.int32),
            pltpu.VMEM((_PAT,), jnp.float32),
            pltpu.VMEM((_LANES,), jnp.float32),
            pltpu.VMEM((_ROWS_PER_W,), jnp.float32),
            pltpu.SemaphoreType.DMA,
            pltpu.SemaphoreType.DMA,
        ],
    )
    return run(x.reshape(-1), table_lin, rowpat, colpat, wpad, bias_v)


# speedup vs baseline: 117.0993x; 1.0103x over previous
import jax
import jax.numpy as jnp
import numpy as np
from jax import lax
from jax.experimental import pallas as pl
from jax.experimental.pallas import tpu as pltpu
from jax.experimental.pallas import tpu_sc as plsc

_NUM_TREES = 600
_EMB_DIM = 3
_VOCAB = _NUM_TREES * 1024
_BATCH = 16384
_LANES = 16

_NW = 32
_NSUB = 16
_ROWS_PER_W = _BATCH // _NW
_CB = 8
_NCHUNK = _ROWS_PER_W // _CB
_ROW_ELEMS = _NUM_TREES * _EMB_DIM
_NVEC = (_ROW_ELEMS + _LANES - 1) // _LANES
_PAT = _NVEC * _LANES
_NBUF = 1

_FCH = 4
_FROW = _VOCAB // (_NW * _FCH)
_FEL = _FROW * _EMB_DIM
_FQ = _FEL // (3 * _LANES)

_f = np.minimum(np.arange(_PAT, dtype=np.int64), _ROW_ELEMS - 1)
_ROWPAT = np.asarray(_f // _EMB_DIM, dtype=np.int32)
_COLPAT = np.asarray(_f % _EMB_DIM, dtype=np.int32)


def _fmt_body(t1d_hbm, tout_hbm, buf1_v, rows3_v):
    wid = lax.axis_index("s") * 2 + lax.axis_index("c")
    lane = lax.iota(jnp.int32, _LANES)
    rpat = [(m * _LANES + lane) // 3 for m in range(3)]
    cpat = [(m * _LANES + lane) % 3 for m in range(3)]
    for h in range(_FCH):
        g = wid * _FCH + h
        pltpu.sync_copy(t1d_hbm.at[pl.ds(g * _FEL, _FEL)], buf1_v)

        def fill_q(q, carry):
            for m in range(3):
                v = buf1_v[pl.ds((q * 3 + m) * _LANES, _LANES)]
                plsc.store_scatter(rows3_v, [q * _LANES + rpat[m], cpat[m]], v)
            return carry

        lax.fori_loop(0, _FQ, fill_q, 0)
        pltpu.sync_copy(rows3_v, tout_hbm.at[pl.ds(g * _FROW, _FROW), :])


def _xfmt_body(x2d_hbm, xf_hbm, blk_v, out1_v):
    wid = lax.axis_index("s") * 2 + lax.axis_index("c")
    lane = lax.iota(jnp.int32, _LANES)
    hi8 = lane >= 8
    base = wid * _ROWS_PER_W

    def blk_body(b, carry):
        row0 = base + b * 32
        pltpu.sync_copy(x2d_hbm.at[pl.ds(row0, 32), :], blk_v)

        def row_body(r, cr):
            rb = r * _NUM_TREES
            rvec = jnp.full((_LANES,), r, dtype=jnp.int32)
            for k in range(37):
                v = plsc.load_gather(blk_v, [rvec, k * _LANES + lane])
                plsc.store_scatter(out1_v, [rb + k * _LANES + lane], v)
            v = plsc.load_gather(blk_v, [rvec, 584 + lane])
            plsc.store_scatter(out1_v, [rb + 584 + lane], v, mask=hi8)
            return cr

        lax.fori_loop(0, 32, row_body, 0)
        pltpu.sync_copy(out1_v,
                        xf_hbm.at[pl.ds(row0 * _NUM_TREES, 32 * _NUM_TREES)])
        return carry

    lax.fori_loop(0, _ROWS_PER_W // 32, blk_body, 0)


def _body(xf_hbm, spt, rowpat_hbm, colpat_hbm, wpad_hbm, bias_hbm,
          out_hbm,
          idx_v, rows_v, rowpat_v, colpat_v, wpad_v,
          bias_v, outbuf_v, sem0, sem1):
    sems = (sem0, sem1)
    wid = lax.axis_index("s") * 2 + lax.axis_index("c")
    lane = lax.iota(jnp.int32, _LANES)
    lane0 = lane == 0

    pltpu.sync_copy(rowpat_hbm, rowpat_v)
    pltpu.sync_copy(colpat_hbm, colpat_v)
    pltpu.sync_copy(wpad_hbm, wpad_v)
    pltpu.sync_copy(bias_hbm, bias_v)
    base_row = wid * _ROWS_PER_W

    def fire(c, p):
        pltpu.sync_copy(
            xf_hbm.at[pl.ds((base_row + c * _CB) * _NUM_TREES,
                            _CB * _NUM_TREES)],
            idx_v.at[p])
        pltpu.make_async_copy(
            spt.at[idx_v.at[p]], rows_v.at[p], sems[p]).start()

    def drain(p):
        pltpu.make_async_copy(
            spt.at[idx_v.at[p]], rows_v.at[p], sems[p]).wait()

    def compute(c, p):
        bias = bias_v[...]
        rows_p = rows_v.at[p]

        def j_body(j, accs):
            jo = j * _LANES
            rp = rowpat_v[pl.ds(jo, _LANES)]
            cv = colpat_v[pl.ds(jo, _LANES)]
            w = wpad_v[pl.ds(jo, _LANES)]
            new = []
            for r in range(_CB):
                g = plsc.load_gather(rows_p, [rp + (r * _NUM_TREES), cv])
                new.append(accs[r] + g * w)
            return tuple(new)

        accs = lax.fori_loop(0, _NVEC, j_body,
                             tuple(bias for _ in range(_CB)))
        for r in range(_CB):
            s = jnp.sum(accs[r])
            pos = jnp.full((_LANES,), c * _CB + r, dtype=jnp.int32)
            val = jnp.full((_LANES,), s, dtype=jnp.float32)
            plsc.store_scatter(outbuf_v, [pos], val, mask=lane0)

    for p in range(_NBUF):
        fire(p, p)

    def outer(c0, carry):
        for p in range(_NBUF):
            c = c0 + p
            drain(p)
            compute(c, p)

            @pl.when(c + _NBUF < _NCHUNK)
            def _():
                fire(c + _NBUF, p)
        return carry

    lax.fori_loop(0, _NCHUNK // _NBUF, lambda i, cr: outer(i * _NBUF, cr), 0)
    pltpu.sync_copy(outbuf_v, out_hbm.at[pl.ds(base_row, _ROWS_PER_W)])


def kernel(x, emb_table, lin_weight, out_bias):
    tflat = emb_table.reshape(-1)
    wpad = jnp.concatenate(
        [lin_weight.reshape(-1), jnp.zeros((_PAT - _ROW_ELEMS,), jnp.float32)])
    bias_v = jnp.zeros((_LANES,), jnp.float32).at[0].set(out_bias)
    rowpat = jnp.asarray(_ROWPAT)
    colpat = jnp.asarray(_COLPAT)

    mesh = plsc.VectorSubcoreMesh(core_axis_name="c", subcore_axis_name="s")
    fmt = pl.kernel(
        _fmt_body,
        mesh=mesh,
        compiler_params=pltpu.CompilerParams(needs_layout_passes=False,
                                             use_tc_tiling_on_sc=False),
        out_type=pltpu.HBM((_VOCAB, _EMB_DIM), jnp.float32),
        scratch_types=[
            pltpu.VMEM((_FEL,), jnp.float32),
            pltpu.VMEM((_FROW, _EMB_DIM), jnp.float32),
        ],
    )
    table_lin = fmt(tflat)
    xfmt = pl.kernel(
        _xfmt_body,
        mesh=mesh,
        compiler_params=pltpu.CompilerParams(needs_layout_passes=False,
                                             use_tc_tiling_on_sc=True),
        out_type=pltpu.HBM((_BATCH * _NUM_TREES,), jnp.int32),
        scratch_types=[
            pltpu.VMEM((32, _NUM_TREES), jnp.int32),
            pltpu.VMEM((32 * _NUM_TREES,), jnp.int32),
        ],
    )
    xf = xfmt(x)
    run = pl.kernel(
        _body,
        mesh=mesh,
        compiler_params=pltpu.CompilerParams(needs_layout_passes=False,
                                             use_tc_tiling_on_sc=False),
        out_type=jax.ShapeDtypeStruct((_BATCH,), jnp.float32),
        scratch_types=[
            pltpu.VMEM((_NBUF, _CB * _NUM_TREES), jnp.int32),
            pltpu.VMEM((_NBUF, _CB * _NUM_TREES, _EMB_DIM), jnp.float32),
            pltpu.VMEM((_PAT,), jnp.int32),
            pltpu.VMEM((_PAT,), jnp.int32),
            pltpu.VMEM((_PAT,), jnp.float32),
            pltpu.VMEM((_LANES,), jnp.float32),
            pltpu.VMEM((_ROWS_PER_W,), jnp.float32),
            pltpu.SemaphoreType.DMA,
            pltpu.SemaphoreType.DMA,
        ],
    )
    return run(xf, table_lin, rowpat, colpat, wpad, bias_v)
